# Initial kernel scaffold; baseline (speedup 1.0000x reference)
import jax
import jax.numpy as jnp
from jax.experimental import pallas as pl


def kernel(concept_ids, relations, head_ids, tail_ids, concept_table, relation_table, W_s, W_n, W_r):
    raise NotImplementedError("write your pallas kernel here")


# speedup vs baseline: 2184.1574x; 2184.1574x over previous
import functools

import jax
import jax.numpy as jnp
from jax import lax
from jax.experimental import pallas as pl
from jax.experimental.pallas import tpu as pltpu
from jax.experimental.pallas import tpu_sc as plsc

B, M, Mt, E = 2, 10000, 160000, 128
NC, NS = 2, 16
NW = NC * NS
K = 80
EPS = Mt // NS
NCH = EPS // K
GROWS = 640
MP = 10240
RPS = MP // NS

_mesh = plsc.VectorSubcoreMesh(core_axis_name="c", subcore_axis_name="s")


@functools.partial(
    pl.kernel,
    out_type=jax.ShapeDtypeStruct((NW * GROWS, E), jnp.float32),
    mesh=_mesh,
    scratch_types=[
        pltpu.VMEM((128,), jnp.int32),
        pltpu.VMEM((128, E), jnp.float32),
        pltpu.SemaphoreType.DMA,
    ],
)
def _sc_gather_c(table_hbm, ids_hbm, out_hbm, idx_v, rows_v, sem):
    w = lax.axis_index("s") * NC + lax.axis_index("c")
    for k in range(GROWS // 128):
        base = w * GROWS + k * 128
        pltpu.sync_copy(ids_hbm.at[pl.ds(base, 128)], idx_v)
        pltpu.async_copy(table_hbm.at[idx_v], rows_v, sem).wait()
        pltpu.sync_copy(rows_v, out_hbm.at[pl.ds(base, 128)])


@functools.partial(
    pl.kernel,
    out_type=jax.ShapeDtypeStruct((B, MP, E), jnp.float32),
    mesh=_mesh,
    scratch_types=[
        pltpu.VMEM((K,), jnp.int32),
        pltpu.VMEM((K,), jnp.int32),
        pltpu.VMEM((K,), jnp.int32),
        pltpu.VMEM((K,), jnp.int32),
        pltpu.VMEM((K,), jnp.int32),
        pltpu.VMEM((K, E), jnp.float32),
        pltpu.VMEM((K, E), jnp.float32),
        pltpu.VMEM((K, E), jnp.float32),
        pltpu.SemaphoreType.DMA,
        pltpu.VMEM_SHARED((MP, E), jnp.float32),
    ],
)
def _sc_aggregate(c_hbm, head_hbm, tail_hbm, headg_hbm, tailg_hbm, rel_hbm, negrel_hbm,
                  zacc_hbm,
                  acc_out,
                  idx_h, idx_t, idx_r, idx_hg, idx_tg,
                  buf_h, buf_t, buf_r, sem,
                  acc_sh):
    c = lax.axis_index("c")
    s = lax.axis_index("s")
    pltpu.sync_copy(zacc_hbm, acc_sh.at[pl.ds(s * RPS, RPS)])
    plsc.subcore_barrier()

    @pl.loop(0, NCH)
    def chunk(i):
        base = c * Mt + s * EPS + i * K
        pltpu.sync_copy(head_hbm.at[pl.ds(base, K)], idx_h)
        pltpu.sync_copy(tail_hbm.at[pl.ds(base, K)], idx_t)
        pltpu.sync_copy(headg_hbm.at[pl.ds(base, K)], idx_hg)
        pltpu.sync_copy(tailg_hbm.at[pl.ds(base, K)], idx_tg)
        pltpu.sync_copy(rel_hbm.at[pl.ds(base, K)], idx_r)
        pltpu.async_copy(c_hbm.at[idx_hg], buf_h, sem).wait()
        pltpu.async_copy(c_hbm.at[idx_tg], buf_t, sem).wait()
        pltpu.async_copy(negrel_hbm.at[idx_r], buf_r, sem).wait()
        pltpu.sync_copy(buf_h, acc_sh.at[idx_t], add=True)
        pltpu.sync_copy(buf_t, acc_sh.at[idx_h], add=True)
        pltpu.sync_copy(buf_r, acc_sh.at[idx_t], add=True)
        pltpu.sync_copy(buf_r, acc_sh.at[idx_h], add=True)

    plsc.subcore_barrier()
    pltpu.sync_copy(acc_sh.at[pl.ds(s * RPS, RPS)], acc_out.at[c, pl.ds(s * RPS, RPS)])


CROWS = 640
CRPS = CROWS // NS


@functools.partial(
    pl.kernel,
    out_type=jax.ShapeDtypeStruct((B, CROWS, E), jnp.float32),
    mesh=_mesh,
    scratch_types=[
        pltpu.VMEM((K,), jnp.int32),
        pltpu.VMEM((K,), jnp.int32),
        pltpu.VMEM((K,), jnp.int32),
        pltpu.VMEM((K,), jnp.int32),
        pltpu.VMEM((K, E), jnp.float32),
        pltpu.VMEM((K, E), jnp.float32),
        pltpu.SemaphoreType.DMA,
        pltpu.VMEM_SHARED((CROWS, E), jnp.float32),
    ],
)
def _sc_count(tdiv_hbm, tmod_hbm, hdiv_hbm, hmod_hbm, pat_hbm, zcnt_hbm,
              cnt_out,
              idx_td, idx_tm, idx_hd, idx_hm, buf_pt, buf_ph, sem,
              cnt_sh):
    c = lax.axis_index("c")
    s = lax.axis_index("s")
    pltpu.sync_copy(zcnt_hbm, cnt_sh.at[pl.ds(s * CRPS, CRPS)])
    plsc.subcore_barrier()

    @pl.loop(0, NCH)
    def chunk(i):
        base = c * Mt + s * EPS + i * K
        pltpu.sync_copy(tdiv_hbm.at[pl.ds(base, K)], idx_td)
        pltpu.sync_copy(tmod_hbm.at[pl.ds(base, K)], idx_tm)
        pltpu.sync_copy(hdiv_hbm.at[pl.ds(base, K)], idx_hd)
        pltpu.sync_copy(hmod_hbm.at[pl.ds(base, K)], idx_hm)
        pltpu.async_copy(pat_hbm.at[idx_tm], buf_pt, sem).wait()
        pltpu.async_copy(pat_hbm.at[idx_hm], buf_ph, sem).wait()
        pltpu.sync_copy(buf_pt, cnt_sh.at[idx_td], add=True)
        pltpu.sync_copy(buf_ph, cnt_sh.at[idx_hd], add=True)

    plsc.subcore_barrier()
    pltpu.sync_copy(cnt_sh.at[pl.ds(s * CRPS, CRPS)], cnt_out.at[c, pl.ds(s * CRPS, CRPS)])


def _dense_body(c_ref, acc_ref, cnt_ref, ws_ref, wn_ref, o_ref):
    cnt = jnp.maximum(cnt_ref[:, 0:1], 1.0)
    upd = acc_ref[...] * (1.0 / cnt)
    x = jnp.dot(c_ref[...], ws_ref[...], preferred_element_type=jnp.float32)
    x = x + jnp.dot(upd, wn_ref[...], preferred_element_type=jnp.float32)
    o_ref[...] = jnp.maximum(x, 0.0)


def _relhid_body(rt_ref, wr_ref, o_ref):
    o_ref[...] = jnp.dot(rt_ref[...], wr_ref[...], preferred_element_type=jnp.float32)


@functools.partial(
    pl.kernel,
    out_type=jax.ShapeDtypeStruct((B, Mt, 3 * E), jnp.float32),
    mesh=_mesh,
    scratch_types=[
        pltpu.VMEM((K,), jnp.int32),
        pltpu.VMEM((K,), jnp.int32),
        pltpu.VMEM((K,), jnp.int32),
        pltpu.VMEM((K, E), jnp.float32),
        pltpu.VMEM((K, E), jnp.float32),
        pltpu.VMEM((K, E), jnp.float32),
        pltpu.SemaphoreType.DMA,
    ],
)
def _sc_assemble(ch_hbm, relhid_hbm, headg_hbm, tailg_hbm, rel_hbm, out_hbm,
                 idx_hg, idx_tg, idx_r,
                 buf_h, buf_r, buf_t, sem):
    c = lax.axis_index("c")
    s = lax.axis_index("s")

    @pl.loop(0, NCH)
    def chunk(i):
        ebase = s * EPS + i * K
        base = c * Mt + ebase
        pltpu.sync_copy(headg_hbm.at[pl.ds(base, K)], idx_hg)
        pltpu.sync_copy(tailg_hbm.at[pl.ds(base, K)], idx_tg)
        pltpu.sync_copy(rel_hbm.at[pl.ds(base, K)], idx_r)
        pltpu.async_copy(ch_hbm.at[idx_hg], buf_h, sem).wait()
        pltpu.async_copy(relhid_hbm.at[idx_r], buf_r, sem).wait()
        pltpu.async_copy(ch_hbm.at[idx_tg], buf_t, sem).wait()
        pltpu.sync_copy(buf_h, out_hbm.at[c, pl.ds(ebase, K), pl.ds(0, E)])
        pltpu.sync_copy(buf_r, out_hbm.at[c, pl.ds(ebase, K), pl.ds(E, E)])
        pltpu.sync_copy(buf_t, out_hbm.at[c, pl.ds(ebase, K), pl.ds(2 * E, E)])


def kernel(concept_ids, relations, head_ids, tail_ids, concept_table,
           relation_table, W_s, W_n, W_r):
    i32 = jnp.int32
    ci = concept_ids.astype(i32).reshape(-1)
    ci_pad = jnp.concatenate([ci, jnp.zeros((NW * GROWS - B * M,), i32)])
    rel = relations.astype(i32).reshape(-1)
    hd = head_ids.astype(i32).reshape(-1)
    tl = tail_ids.astype(i32).reshape(-1)
    boff = (jnp.arange(B, dtype=i32)[:, None] * M).repeat(Mt, axis=1).reshape(-1)
    hd_g = hd + boff
    tl_g = tl + boff

    c_pad = _sc_gather_c(concept_table, ci_pad)

    zacc = jnp.zeros((RPS, E), jnp.float32)
    accp = _sc_aggregate(c_pad, hd, tl, hd_g, tl_g, rel, -relation_table, zacc)
    acc = accp[:, :M]

    t_div = tl >> 4
    t_mod = tl & 15
    h_div = hd >> 4
    h_mod = hd & 15
    pat = (jnp.arange(E, dtype=i32)[None, :] // 8 ==
           jnp.arange(16, dtype=i32)[:, None]).astype(jnp.float32)
    zcnt = jnp.zeros((CRPS, E), jnp.float32)
    cntp = _sc_count(t_div, t_mod, h_div, h_mod, pat, zcnt)
    cnt8 = cntp.reshape(B, MP, 8)[:, :M]

    ws_t = W_s[-1].T
    wn_t = W_n[-1].T
    c2 = c_pad[:B * M]
    ch = pl.pallas_call(
        _dense_body,
        grid=(10,),
        in_specs=[
            pl.BlockSpec((B * M // 10, E), lambda i: (i, 0)),
            pl.BlockSpec((B * M // 10, E), lambda i: (i, 0)),
            pl.BlockSpec((B * M // 10, 8), lambda i: (i, 0)),
            pl.BlockSpec((E, E), lambda i: (0, 0)),
            pl.BlockSpec((E, E), lambda i: (0, 0)),
        ],
        out_specs=pl.BlockSpec((B * M // 10, E), lambda i: (i, 0)),
        out_shape=jax.ShapeDtypeStruct((B * M, E), jnp.float32),
    )(c2, acc.reshape(B * M, E), cnt8.reshape(B * M, 8), ws_t, wn_t)

    relhid = pl.pallas_call(
        _relhid_body,
        out_shape=jax.ShapeDtypeStruct((relation_table.shape[0], E), jnp.float32),
    )(relation_table, W_r[-1].T)

    return _sc_assemble(ch, relhid, hd_g, tl_g, rel)


# speedup vs baseline: 2393.5590x; 1.0959x over previous
import functools

import jax
import jax.numpy as jnp
from jax import lax
from jax.experimental import pallas as pl
from jax.experimental.pallas import tpu as pltpu
from jax.experimental.pallas import tpu_sc as plsc

B, M, Mt, E = 2, 10000, 160000, 128
NC, NS = 2, 16
NW = NC * NS
K = 80
EPS = Mt // NS
NCH = EPS // K
GROWS = 640
MP = 10240
RPS = MP // NS

_mesh = plsc.VectorSubcoreMesh(core_axis_name="c", subcore_axis_name="s")


@functools.partial(
    pl.kernel,
    out_type=jax.ShapeDtypeStruct((NW * GROWS, E), jnp.float32),
    mesh=_mesh,
    scratch_types=[
        pltpu.VMEM((128,), jnp.int32),
        pltpu.VMEM((128, E), jnp.float32),
        pltpu.SemaphoreType.DMA,
    ],
)
def _sc_gather_c(table_hbm, ids_hbm, out_hbm, idx_v, rows_v, sem):
    w = lax.axis_index("s") * NC + lax.axis_index("c")
    for k in range(GROWS // 128):
        base = w * GROWS + k * 128
        pltpu.sync_copy(ids_hbm.at[pl.ds(base, 128)], idx_v)
        pltpu.async_copy(table_hbm.at[idx_v], rows_v, sem).wait()
        pltpu.sync_copy(rows_v, out_hbm.at[pl.ds(base, 128)])


@functools.partial(
    pl.kernel,
    out_type=jax.ShapeDtypeStruct((B, MP, E), jnp.float32),
    mesh=_mesh,
    scratch_types=[
        pltpu.VMEM((K,), jnp.int32),
        pltpu.VMEM((K,), jnp.int32),
        pltpu.VMEM((K,), jnp.int32),
        pltpu.VMEM((K,), jnp.int32),
        pltpu.VMEM((K,), jnp.int32),
        pltpu.VMEM((K, E), jnp.float32),
        pltpu.VMEM((K, E), jnp.float32),
        pltpu.VMEM((K, E), jnp.float32),
        pltpu.SemaphoreType.DMA,
        pltpu.VMEM_SHARED((MP, E), jnp.float32),
    ],
)
def _sc_aggregate(c_hbm, head_hbm, tail_hbm, headg_hbm, tailg_hbm, rel_hbm, negrel_hbm,
                  zacc_hbm,
                  acc_out,
                  idx_h, idx_t, idx_r, idx_hg, idx_tg,
                  buf_h, buf_t, buf_r, sem,
                  acc_sh):
    c = lax.axis_index("c")
    s = lax.axis_index("s")
    pltpu.sync_copy(zacc_hbm, acc_sh.at[pl.ds(s * RPS, RPS)])
    plsc.subcore_barrier()

    @pl.loop(0, NCH)
    def chunk(i):
        base = c * Mt + s * EPS + i * K
        l1 = pltpu.async_copy(head_hbm.at[pl.ds(base, K)], idx_h, sem)
        l2 = pltpu.async_copy(tail_hbm.at[pl.ds(base, K)], idx_t, sem)
        l3 = pltpu.async_copy(headg_hbm.at[pl.ds(base, K)], idx_hg, sem)
        l4 = pltpu.async_copy(tailg_hbm.at[pl.ds(base, K)], idx_tg, sem)
        l5 = pltpu.async_copy(rel_hbm.at[pl.ds(base, K)], idx_r, sem)
        l1.wait(); l2.wait(); l3.wait(); l4.wait(); l5.wait()
        g1 = pltpu.async_copy(c_hbm.at[idx_hg], buf_h, sem)
        g2 = pltpu.async_copy(c_hbm.at[idx_tg], buf_t, sem)
        g3 = pltpu.async_copy(negrel_hbm.at[idx_r], buf_r, sem)
        g1.wait(); g2.wait(); g3.wait()
        s1 = pltpu.async_copy(buf_h, acc_sh.at[idx_t], sem, add=True)
        s2 = pltpu.async_copy(buf_t, acc_sh.at[idx_h], sem, add=True)
        s3 = pltpu.async_copy(buf_r, acc_sh.at[idx_t], sem, add=True)
        s4 = pltpu.async_copy(buf_r, acc_sh.at[idx_h], sem, add=True)
        s1.wait(); s2.wait(); s3.wait(); s4.wait()

    plsc.subcore_barrier()
    pltpu.sync_copy(acc_sh.at[pl.ds(s * RPS, RPS)], acc_out.at[c, pl.ds(s * RPS, RPS)])


CROWS = 640
CRPS = CROWS // NS


@functools.partial(
    pl.kernel,
    out_type=jax.ShapeDtypeStruct((B, CROWS, E), jnp.float32),
    mesh=_mesh,
    scratch_types=[
        pltpu.VMEM((K,), jnp.int32),
        pltpu.VMEM((K,), jnp.int32),
        pltpu.VMEM((K,), jnp.int32),
        pltpu.VMEM((K,), jnp.int32),
        pltpu.VMEM((K, E), jnp.float32),
        pltpu.VMEM((K, E), jnp.float32),
        pltpu.SemaphoreType.DMA,
        pltpu.VMEM_SHARED((CROWS, E), jnp.float32),
    ],
)
def _sc_count(tdiv_hbm, tmod_hbm, hdiv_hbm, hmod_hbm, pat_hbm, zcnt_hbm,
              cnt_out,
              idx_td, idx_tm, idx_hd, idx_hm, buf_pt, buf_ph, sem,
              cnt_sh):
    c = lax.axis_index("c")
    s = lax.axis_index("s")
    pltpu.sync_copy(zcnt_hbm, cnt_sh.at[pl.ds(s * CRPS, CRPS)])
    plsc.subcore_barrier()

    @pl.loop(0, NCH)
    def chunk(i):
        base = c * Mt + s * EPS + i * K
        l1 = pltpu.async_copy(tdiv_hbm.at[pl.ds(base, K)], idx_td, sem)
        l2 = pltpu.async_copy(tmod_hbm.at[pl.ds(base, K)], idx_tm, sem)
        l3 = pltpu.async_copy(hdiv_hbm.at[pl.ds(base, K)], idx_hd, sem)
        l4 = pltpu.async_copy(hmod_hbm.at[pl.ds(base, K)], idx_hm, sem)
        l1.wait(); l2.wait(); l3.wait(); l4.wait()
        g1 = pltpu.async_copy(pat_hbm.at[idx_tm], buf_pt, sem)
        g2 = pltpu.async_copy(pat_hbm.at[idx_hm], buf_ph, sem)
        g1.wait(); g2.wait()
        s1 = pltpu.async_copy(buf_pt, cnt_sh.at[idx_td], sem, add=True)
        s2 = pltpu.async_copy(buf_ph, cnt_sh.at[idx_hd], sem, add=True)
        s1.wait(); s2.wait()

    plsc.subcore_barrier()
    pltpu.sync_copy(cnt_sh.at[pl.ds(s * CRPS, CRPS)], cnt_out.at[c, pl.ds(s * CRPS, CRPS)])


def _dense_body(c_ref, acc_ref, cnt_ref, ws_ref, wn_ref, o_ref):
    cnt = jnp.maximum(cnt_ref[:, 0:1], 1.0)
    upd = acc_ref[...] * (1.0 / cnt)
    x = jnp.dot(c_ref[...], ws_ref[...], preferred_element_type=jnp.float32)
    x = x + jnp.dot(upd, wn_ref[...], preferred_element_type=jnp.float32)
    o_ref[...] = jnp.maximum(x, 0.0)


def _relhid_body(rt_ref, wr_ref, o_ref):
    o_ref[...] = jnp.dot(rt_ref[...], wr_ref[...], preferred_element_type=jnp.float32)


@functools.partial(
    pl.kernel,
    out_type=jax.ShapeDtypeStruct((B, Mt, 3 * E), jnp.float32),
    mesh=_mesh,
    scratch_types=[
        pltpu.VMEM((K,), jnp.int32),
        pltpu.VMEM((K,), jnp.int32),
        pltpu.VMEM((K,), jnp.int32),
        pltpu.VMEM((K, E), jnp.float32),
        pltpu.VMEM((K, E), jnp.float32),
        pltpu.VMEM((K, E), jnp.float32),
        pltpu.SemaphoreType.DMA,
    ],
)
def _sc_assemble(ch_hbm, relhid_hbm, headg_hbm, tailg_hbm, rel_hbm, out_hbm,
                 idx_hg, idx_tg, idx_r,
                 buf_h, buf_r, buf_t, sem):
    c = lax.axis_index("c")
    s = lax.axis_index("s")

    @pl.loop(0, NCH)
    def chunk(i):
        ebase = s * EPS + i * K
        base = c * Mt + ebase
        l1 = pltpu.async_copy(headg_hbm.at[pl.ds(base, K)], idx_hg, sem)
        l2 = pltpu.async_copy(tailg_hbm.at[pl.ds(base, K)], idx_tg, sem)
        l3 = pltpu.async_copy(rel_hbm.at[pl.ds(base, K)], idx_r, sem)
        l1.wait(); l2.wait(); l3.wait()
        g1 = pltpu.async_copy(ch_hbm.at[idx_hg], buf_h, sem)
        g2 = pltpu.async_copy(relhid_hbm.at[idx_r], buf_r, sem)
        g3 = pltpu.async_copy(ch_hbm.at[idx_tg], buf_t, sem)
        g1.wait(); g2.wait(); g3.wait()
        w1 = pltpu.async_copy(buf_h, out_hbm.at[c, pl.ds(ebase, K), pl.ds(0, E)], sem)
        w2 = pltpu.async_copy(buf_r, out_hbm.at[c, pl.ds(ebase, K), pl.ds(E, E)], sem)
        w3 = pltpu.async_copy(buf_t, out_hbm.at[c, pl.ds(ebase, K), pl.ds(2 * E, E)], sem)
        w1.wait(); w2.wait(); w3.wait()


def kernel(concept_ids, relations, head_ids, tail_ids, concept_table,
           relation_table, W_s, W_n, W_r):
    i32 = jnp.int32
    ci = concept_ids.astype(i32).reshape(-1)
    ci_pad = jnp.concatenate([ci, jnp.zeros((NW * GROWS - B * M,), i32)])
    rel = relations.astype(i32).reshape(-1)
    hd = head_ids.astype(i32).reshape(-1)
    tl = tail_ids.astype(i32).reshape(-1)
    boff = (jnp.arange(B, dtype=i32)[:, None] * M).repeat(Mt, axis=1).reshape(-1)
    hd_g = hd + boff
    tl_g = tl + boff

    c_pad = _sc_gather_c(concept_table, ci_pad)

    zacc = jnp.zeros((RPS, E), jnp.float32)
    accp = _sc_aggregate(c_pad, hd, tl, hd_g, tl_g, rel, -relation_table, zacc)
    acc = accp[:, :M]

    t_div = tl >> 4
    t_mod = tl & 15
    h_div = hd >> 4
    h_mod = hd & 15
    pat = (jnp.arange(E, dtype=i32)[None, :] // 8 ==
           jnp.arange(16, dtype=i32)[:, None]).astype(jnp.float32)
    zcnt = jnp.zeros((CRPS, E), jnp.float32)
    cntp = _sc_count(t_div, t_mod, h_div, h_mod, pat, zcnt)
    cnt8 = cntp.reshape(B, MP, 8)[:, :M]

    ws_t = W_s[-1].T
    wn_t = W_n[-1].T
    c2 = c_pad[:B * M]
    ch = pl.pallas_call(
        _dense_body,
        grid=(10,),
        in_specs=[
            pl.BlockSpec((B * M // 10, E), lambda i: (i, 0)),
            pl.BlockSpec((B * M // 10, E), lambda i: (i, 0)),
            pl.BlockSpec((B * M // 10, 8), lambda i: (i, 0)),
            pl.BlockSpec((E, E), lambda i: (0, 0)),
            pl.BlockSpec((E, E), lambda i: (0, 0)),
        ],
        out_specs=pl.BlockSpec((B * M // 10, E), lambda i: (i, 0)),
        out_shape=jax.ShapeDtypeStruct((B * M, E), jnp.float32),
    )(c2, acc.reshape(B * M, E), cnt8.reshape(B * M, 8), ws_t, wn_t)

    relhid = pl.pallas_call(
        _relhid_body,
        out_shape=jax.ShapeDtypeStruct((relation_table.shape[0], E), jnp.float32),
    )(relation_table, W_r[-1].T)

    return _sc_assemble(ch, relhid, hd_g, tl_g, rel)
